# baseline (speedup 1.0000x reference)
import functools

import jax
import jax.numpy as jnp
from jax import lax
from jax.experimental import pallas as pl
from jax.experimental.pallas import tpu as pltpu
from jax.experimental.pallas import tpu_sc as plsc

N = 10000
E = 320000
IN_C = 128
OUT_C = 128
HEADS = 4
KC = 5
T = 4
HOC = OUT_C * HEADS

EP = 327680
PAD = EP - E
DN = 40960
NB = 400
NBLK = N // NB

_f32 = jnp.float32
_i32 = jnp.int32


def _phase_a_body(x_ref, w_ref, atta_ref, attb_ref, bias_ref,
                  xtg_ref, sd_ref, ss_ref, init_ref):
    xb = x_ref[...]
    for t in range(T):
        y = jnp.dot(xb, w_ref[t], preferred_element_type=_f32)
        sds = []
        sss = []
        for h in range(HEADS):
            yh = y[:, h * OUT_C:(h + 1) * OUT_C]
            xtg_ref[h, t] = yh
            sds.append(jnp.dot(yh, atta_ref[h], preferred_element_type=_f32))
            sss.append(jnp.dot(yh, attb_ref[h], preferred_element_type=_f32))
            if t == 0:
                init_ref[h] = yh + bias_ref[:, h * OUT_C:(h + 1) * OUT_C]
        sd_ref[t] = jnp.concatenate(sds, axis=1)
        ss_ref[t] = jnp.concatenate(sss, axis=1)


def _phase_a(x, W, attA, attB, bias2):
    return pl.pallas_call(
        _phase_a_body,
        grid=(NBLK,),
        in_specs=[
            pl.BlockSpec((NB, IN_C), lambda i: (i, 0)),
            pl.BlockSpec((T, IN_C, HOC), lambda i: (0, 0, 0)),
            pl.BlockSpec((HEADS, OUT_C, 8), lambda i: (0, 0, 0)),
            pl.BlockSpec((HEADS, OUT_C, 8), lambda i: (0, 0, 0)),
            pl.BlockSpec((1, HOC), lambda i: (0, 0)),
        ],
        out_specs=[
            pl.BlockSpec((HEADS, T, NB, OUT_C), lambda i: (0, 0, i, 0)),
            pl.BlockSpec((T, NB, 32), lambda i: (0, i, 0)),
            pl.BlockSpec((T, NB, 32), lambda i: (0, i, 0)),
            pl.BlockSpec((HEADS, NB, OUT_C), lambda i: (0, i, 0)),
        ],
        out_shape=[
            jax.ShapeDtypeStruct((HEADS, T, N, OUT_C), _f32),
            jax.ShapeDtypeStruct((T, N, 32), _f32),
            jax.ShapeDtypeStruct((T, N, 32), _f32),
            jax.ShapeDtypeStruct((HEADS, N, OUT_C), _f32),
        ],
    )(x, W, attA, attB, bias2)


_B_CHUNK = 128
_B_CHUNKS = EP // 32 // _B_CHUNK


def _b_set():
    return [
        pltpu.VMEM((_B_CHUNK,), _i32),
        pltpu.VMEM((_B_CHUNK,), _i32),
        pltpu.VMEM((_B_CHUNK,), _i32),
        pltpu.VMEM((_B_CHUNK,), _i32),
        pltpu.VMEM((_B_CHUNK,), _i32),
        pltpu.VMEM((_B_CHUNK, 32), _f32),
        pltpu.VMEM((_B_CHUNK, 32), _f32),
        pltpu.VMEM((HEADS, _B_CHUNK), _f32),
        pltpu.VMEM((_B_CHUNK, 16), _f32),
        pltpu.SemaphoreType.DMA,
        pltpu.SemaphoreType.DMA,
        pltpu.SemaphoreType.DMA,
    ]


@functools.cache
def _phase_b_built():
    mesh = plsc.VectorSubcoreMesh(core_axis_name="c", subcore_axis_name="s")
    return functools.partial(
        pl.kernel, mesh=mesh,
        compiler_params=pltpu.CompilerParams(
            needs_layout_passes=False, use_tc_tiling_on_sc=False),
        out_type=[
            jax.ShapeDtypeStruct((EP // 128, HEADS, 128), _f32),
            jax.ShapeDtypeStruct((2, DN, 16), _f32),
        ],
        scratch_types=_b_set() + _b_set() + [
            pltpu.VMEM((16, 16), _f32),
            pltpu.VMEM((DN // 16, 16), _f32),
            pltpu.VMEM_SHARED((DN, 16), _f32),
        ],
    )(_phase_b)


def _phase_b(src_hbm, dst_hbm, typ_hbm, sd_hbm, ss_hbm, probs_hbm,
             ex_hbm, denom_hbm, *refs):
    sets = (refs[0:12], refs[12:24])
    probv, zbuf, denom_sh = refs[24], refs[25], refs[26]
    cid = lax.axis_index("c")
    sid = lax.axis_index("s")
    w = cid * 16 + sid
    z16 = jnp.zeros((16,), _f32)
    rows_per_tile = DN // 16
    wbase = w * (_B_CHUNK * _B_CHUNKS)
    iota16 = lax.iota(_i32, 16)

    def zrow(i, _):
        zbuf[i] = z16
        return _
    lax.fori_loop(0, rows_per_tile, zrow, None)
    pltpu.sync_copy(zbuf, denom_sh.at[pl.ds(sid * rows_per_tile, rows_per_tile)])
    for b in range(2):
        scat = sets[b][8]
        for i in range(_B_CHUNK):
            scat[i] = z16
    pltpu.sync_copy(probs_hbm, probv)
    plsc.subcore_barrier()

    def issue_loads(i, b):
        srcv, dstv, typv = sets[b][0:3]
        sem_l = sets[b][9]
        base = wbase + jnp.minimum(i, _B_CHUNKS - 1) * _B_CHUNK
        pltpu.async_copy(src_hbm.at[pl.ds(base, _B_CHUNK)], srcv, sem_l)
        pltpu.async_copy(dst_hbm.at[pl.ds(base, _B_CHUNK)], dstv, sem_l)
        pltpu.async_copy(typ_hbm.at[pl.ds(base, _B_CHUNK)], typv, sem_l)

    def wait_loads(b):
        srcv, dstv, typv = sets[b][0:3]
        sem_l = sets[b][9]
        for buf in (srcv, dstv, typv):
            pltpu.make_async_copy(
                src_hbm.at[pl.ds(0, _B_CHUNK)], buf, sem_l).wait()

    def indices(b):
        srcv, dstv, typv, tsrcv, tdstv = sets[b][0:5]
        for g in range(_B_CHUNK // 16):
            sl = pl.ds(g * 16, 16)
            tt = typv[sl] * N
            tsrcv[sl] = tt + srcv[sl]
            tdstv[sl] = tt + dstv[sl]

    def gather_descs(b):
        tsrcv, tdstv, sdb, ssb = sets[b][3:7]
        sem_g = sets[b][10]
        return [(sd_hbm.at[tdstv], sdb, sem_g),
                (ss_hbm.at[tsrcv], ssb, sem_g)]

    def scatter_desc(b):
        tdstv = sets[b][4]
        scat = sets[b][8]
        sem_w = sets[b][11]
        return scat, denom_sh.at[tdstv], sem_w

    def compute(b):
        sdb, ssb, exb, scat = sets[b][5:9]
        pk = [probv[k] for k in range(KC)]
        for g in range(_B_CHUNK // 16):
            rows = iota16 + g * 16
            for h in range(HEADS):
                acc = jnp.zeros((16,), _f32)
                for k in range(KC):
                    col = jnp.full((16,), h * 8 + k, _i32)
                    v = (plsc.load_gather(sdb, [rows, col])
                         + plsc.load_gather(ssb, [rows, col]))
                    v = jnp.maximum(v, 0.2 * v)
                    acc = acc + pk[k] * v
                ev = jnp.exp(acc)
                exb[h, pl.ds(g * 16, 16)] = ev
                plsc.store_scatter(scat, [rows, jnp.full((16,), h, _i32)], ev)

    def step(i, b):
        wait_loads(1 - b)

        @pl.when(i > 0)
        def _drain():
            sc, dd, sem_w = scatter_desc(1 - b)
            pltpu.make_async_copy(sc, dd, sem_w).wait()

        indices(1 - b)
        for s, d, sem in gather_descs(1 - b):
            pltpu.async_copy(s, d, sem)
        issue_loads(i + 2, b)
        for s, d, sem in gather_descs(b):
            pltpu.make_async_copy(s, d, sem).wait()
        compute(b)
        exb = sets[b][7]
        gc = wbase // 128 + jnp.minimum(i, _B_CHUNKS - 1)
        pltpu.sync_copy(exb, ex_hbm.at[gc])
        sc, dd, sem_w = scatter_desc(b)
        pltpu.async_copy(sc, dd, sem_w, add=True)

    issue_loads(0, 0)
    wait_loads(0)
    indices(0)
    for s, d, sem in gather_descs(0):
        pltpu.async_copy(s, d, sem)
    issue_loads(1, 1)

    def body(j, _):
        step(2 * j, 0)
        step(2 * j + 1, 1)
        return _
    lax.fori_loop(0, _B_CHUNKS // 2, body, None)

    for s, d, sem in gather_descs(0):
        pltpu.make_async_copy(s, d, sem).wait()
    wait_loads(1)
    sc, dd, sem_w = scatter_desc(1)
    pltpu.make_async_copy(sc, dd, sem_w).wait()

    plsc.subcore_barrier()
    pltpu.sync_copy(denom_sh.at[pl.ds(sid * rows_per_tile, rows_per_tile)],
                    denom_hbm.at[cid, pl.ds(sid * rows_per_tile, rows_per_tile)])


def _phase_c_body(d_ref, imp_ref, rec_ref):
    rec_ref[...] = imp_ref[...] / (d_ref[0] + d_ref[1] + 1e-16)


def _phase_c(denom_part, imp_col):
    blk = 2560
    return pl.pallas_call(
        _phase_c_body,
        grid=(DN // blk,),
        in_specs=[
            pl.BlockSpec((2, blk, 16), lambda i: (0, i, 0)),
            pl.BlockSpec((blk, 1), lambda i: (i, 0)),
        ],
        out_specs=pl.BlockSpec((blk, 16), lambda i: (i, 0)),
        out_shape=jax.ShapeDtypeStruct((DN, 16), _f32),
    )(denom_part, imp_col)


_DS = 128
_DR = _DS // 128
_D_NSC = EP // 16 // _DS
_ACC_ROWS = N + 16
_NPT = N // 16


def _d_set():
    return [
        pltpu.VMEM((_DS,), _i32),
        pltpu.VMEM((_DS,), _i32),
        pltpu.VMEM((_DS,), _i32),
        pltpu.VMEM((_DR, 128), _i32),
        pltpu.VMEM((_DR, 128), _i32),
        pltpu.VMEM((_DR, 128), _i32),
        pltpu.VMEM((_DS, 16), _f32),
        pltpu.VMEM((_DS, OUT_C), jnp.bfloat16),
        pltpu.VMEM((_DS,), _f32),
        pltpu.SemaphoreType.DMA,
        pltpu.SemaphoreType.DMA,
        pltpu.SemaphoreType.DMA,
    ]


@functools.cache
def _phase_d_built():
    mesh = plsc.VectorSubcoreMesh(core_axis_name="c", subcore_axis_name="s")
    return functools.partial(
        pl.kernel, mesh=mesh,
        compiler_params=pltpu.CompilerParams(
            needs_layout_passes=False, use_tc_tiling_on_sc=False),
        out_type=jax.ShapeDtypeStruct((HEADS, N, OUT_C), _f32),
        scratch_types=_d_set() + _d_set() + [
            pltpu.VMEM((_DS // 16, 16), _f32),
            pltpu.VMEM((_DS, OUT_C), _f32),
            pltpu.VMEM_SHARED((_ACC_ROWS, OUT_C), _f32),
        ],
    )(_phase_d)


def _phase_d(src_hbm, dst_hbm, typ_hbm, xtg_hbm, rec_hbm, ex_hbm, init_hbm,
             out_hbm, *refs):
    sets = (refs[0:12], refs[12:24])
    cbuf, xtf, acc_sh = refs[24], refs[25], refs[26]
    cid = lax.axis_index("c")
    sid = lax.axis_index("s")
    iota16 = lax.iota(_i32, 16)
    hbase = sid * (_DS * _D_NSC)

    for hh in range(2):
        h = cid * 2 + hh
        pltpu.sync_copy(init_hbm.at[h, pl.ds(sid * _NPT, _NPT)],
                        acc_sh.at[pl.ds(sid * _NPT, _NPT)])
        plsc.subcore_barrier()

        def issue_loads(i, b):
            srcv, dstv, typv = sets[b][0:3]
            sem_l = sets[b][9]
            base = hbase + jnp.minimum(i, _D_NSC - 1) * _DS
            pltpu.async_copy(src_hbm.at[pl.ds(base, _DS)], srcv, sem_l)
            pltpu.async_copy(dst_hbm.at[pl.ds(base, _DS)], dstv, sem_l)
            pltpu.async_copy(typ_hbm.at[pl.ds(base, _DS)], typv, sem_l)

        def wait_loads(b):
            srcv, dstv, typv = sets[b][0:3]
            sem_l = sets[b][9]
            for buf in (srcv, dstv, typv):
                pltpu.make_async_copy(
                    src_hbm.at[pl.ds(0, _DS)], buf, sem_l).wait()

        def indices(b):
            srcv, dstv, typv, tdst2, xsrc2, dst2 = sets[b][0:6]
            for g in range(_DS // 16):
                sl = pl.ds(g * 16, 16)
                r, c = g // 8, g % 8
                cs = pl.ds((g % 8) * 16, 16)
                tt = typv[sl] * N
                d16 = dstv[sl]
                tdst2[r, cs] = tt + d16
                xsrc2[r, cs] = tt + srcv[sl] + h * (T * N)
                dst2[r, cs] = d16

        def gather_descs(i, b):
            tdst2, xsrc2 = sets[b][3:5]
            recb, xtb, exv = sets[b][6:9]
            sem_g = sets[b][10]
            base = hbase + jnp.minimum(i, _D_NSC - 1) * _DS
            out = []
            for r in range(_DR):
                rs = pl.ds(r * 128, 128)
                out.append((rec_hbm.at[tdst2.at[r]], recb.at[rs], sem_g))
                out.append((xtg_hbm.at[xsrc2.at[r]], xtb.at[rs], sem_g))
            gc = (hbase + jnp.minimum(i, _D_NSC - 1) * _DS) // 128
            out.append((ex_hbm.at[gc, h], exv, sem_g))
            return out

        def scatter_descs(b):
            dst2 = sets[b][5]
            sem_s = sets[b][11]
            return [(xtf.at[pl.ds(r * 128, 128)], acc_sh.at[dst2.at[r]])
                    for r in range(_DR)], sem_s

        def compute(b):
            recb, xtb, exv = sets[b][6:9]
            hcol = jnp.full((16,), h, _i32)
            for g in range(_DS // 16):
                rows = iota16 + g * 16
                rc = plsc.load_gather(recb, [rows, hcol])
                cbuf[g] = exv[pl.ds(g * 16, 16)] * rc

            def escale(e2, _c):
                for u in range(2):
                    e = e2 * 2 + u
                    ge = jnp.full((16,), e // 16, _i32)
                    le = jnp.full((16,), e % 16, _i32)
                    ce = plsc.load_gather(cbuf, [ge, le])
                    for q in range(OUT_C // 32):
                        v32 = xtb[e, pl.ds(q * 32, 32)]
                        lo, hi = plsc.unpack(
                            v32, format=plsc.PackFormat.INTERLEAVED)
                        xtf[e, pl.ds(q * 32, 16)] = lo * ce
                        xtf[e, pl.ds(q * 32 + 16, 16)] = hi * ce
                return _c
            lax.fori_loop(0, _DS // 2, escale, None)

        def step(i, b):
            wait_loads(1 - b)

            @pl.when(i > 0)
            def _drain():
                descs, sem_s = scatter_descs(1 - b)
                for s, d in descs:
                    pltpu.make_async_copy(s, d, sem_s).wait()

            indices(1 - b)
            for s, d, sem in gather_descs(i + 1, 1 - b):
                pltpu.async_copy(s, d, sem)
            issue_loads(i + 2, b)
            for s, d, sem in gather_descs(i, b):
                pltpu.make_async_copy(s, d, sem).wait()
            compute(b)
            descs, sem_s = scatter_descs(b)
            for s, d in descs:
                pltpu.async_copy(s, d, sem_s, add=True)

        issue_loads(0, 0)
        wait_loads(0)
        indices(0)
        for s, d, sem in gather_descs(0, 0):
            pltpu.async_copy(s, d, sem)
        issue_loads(1, 1)

        def body(j, _):
            step(2 * j, 0)
            step(2 * j + 1, 1)
            return _
        lax.fori_loop(0, _D_NSC // 2, body, None)

        for s, d, sem in gather_descs(_D_NSC, 0):
            pltpu.make_async_copy(s, d, sem).wait()
        wait_loads(1)
        descs, sem_s = scatter_descs(1)
        for s, d in descs:
            pltpu.make_async_copy(s, d, sem_s).wait()

        plsc.subcore_barrier()
        pltpu.sync_copy(acc_sh.at[pl.ds(sid * _NPT, _NPT)],
                        out_hbm.at[h, pl.ds(sid * _NPT, _NPT)])
        plsc.subcore_barrier()


def kernel(x, edge_index, edge_types, W, att_vectors, confounder_probs,
           edge_importance, bias):
    probs = jax.nn.softmax(confounder_probs)
    probs16 = jnp.broadcast_to(
        jnp.zeros((16,), _f32).at[:KC].set(probs)[:, None], (16, 16))

    attA = jnp.zeros((HEADS, OUT_C, 8), _f32).at[:, :, :KC].set(
        jnp.transpose(att_vectors[:, :, :OUT_C], (1, 2, 0)))
    attB = jnp.zeros((HEADS, OUT_C, 8), _f32).at[:, :, :KC].set(
        jnp.transpose(att_vectors[:, :, OUT_C:], (1, 2, 0)))
    bias2 = bias.reshape(1, HOC)

    xtg, sd3, ss3, out_init = _phase_a(x, W, attA, attB, bias2)
    xtg2 = (xtg.astype(jnp.bfloat16)
            .reshape(HEADS * T * N, OUT_C // 32, 2, 16)
            .swapaxes(2, 3)
            .reshape(HEADS * T * N, OUT_C))
    sd2 = jnp.pad(sd3.reshape(T * N, 32), ((0, DN - T * N), (0, 0)))
    ss2 = jnp.pad(ss3.reshape(T * N, 32), ((0, DN - T * N), (0, 0)))

    srcP = jnp.concatenate([edge_index[0], jnp.zeros((PAD,), _i32)])
    dstP = jnp.concatenate([edge_index[1], jnp.full((PAD,), N, _i32)])
    typP = jnp.concatenate([edge_types, jnp.full((PAD,), T - 1, _i32)])

    ex, denom_part = _phase_b_built()(srcP, dstP, typP, sd2, ss2, probs16)

    imp_col = jnp.concatenate(
        [jnp.repeat(edge_importance, N), jnp.ones((DN - T * N,), _f32)]
    ).reshape(DN, 1)
    rec = _phase_c(denom_part, imp_col)

    out_hd = _phase_d_built()(srcP, dstP, typP, xtg2, rec, ex, out_init)
    return out_hd.transpose(1, 0, 2).reshape(N, HOC)


# speedup vs baseline: 1.2086x; 1.2086x over previous
import functools

import jax
import jax.numpy as jnp
from jax import lax
from jax.experimental import pallas as pl
from jax.experimental.pallas import tpu as pltpu
from jax.experimental.pallas import tpu_sc as plsc

N = 10000
E = 320000
IN_C = 128
OUT_C = 128
HEADS = 4
KC = 5
T = 4
HOC = OUT_C * HEADS

EP = 327680
PAD = EP - E
DN = 40960
NB = 400
NBLK = N // NB

_f32 = jnp.float32
_i32 = jnp.int32


def _phase_a_body(x_ref, w_ref, atta_ref, attb_ref, bias_ref,
                  xtg_ref, sd_ref, ss_ref, init_ref):
    xb = x_ref[...]
    for t in range(T):
        y = jnp.dot(xb, w_ref[t], preferred_element_type=_f32)
        sds = []
        sss = []
        for h in range(HEADS):
            yh = y[:, h * OUT_C:(h + 1) * OUT_C]
            xtg_ref[h, t] = yh
            sds.append(jnp.dot(yh, atta_ref[h], preferred_element_type=_f32))
            sss.append(jnp.dot(yh, attb_ref[h], preferred_element_type=_f32))
            if t == 0:
                init_ref[h] = yh + bias_ref[:, h * OUT_C:(h + 1) * OUT_C]
        sd_ref[t] = jnp.concatenate(sds, axis=1)
        ss_ref[t] = jnp.concatenate(sss, axis=1)


def _phase_a(x, W, attA, attB, bias2):
    return pl.pallas_call(
        _phase_a_body,
        grid=(NBLK,),
        in_specs=[
            pl.BlockSpec((NB, IN_C), lambda i: (i, 0)),
            pl.BlockSpec((T, IN_C, HOC), lambda i: (0, 0, 0)),
            pl.BlockSpec((HEADS, OUT_C, 8), lambda i: (0, 0, 0)),
            pl.BlockSpec((HEADS, OUT_C, 8), lambda i: (0, 0, 0)),
            pl.BlockSpec((1, HOC), lambda i: (0, 0)),
        ],
        out_specs=[
            pl.BlockSpec((HEADS, T, NB, OUT_C), lambda i: (0, 0, i, 0)),
            pl.BlockSpec((T, NB, 32), lambda i: (0, i, 0)),
            pl.BlockSpec((T, NB, 32), lambda i: (0, i, 0)),
            pl.BlockSpec((HEADS, NB, OUT_C), lambda i: (0, i, 0)),
        ],
        out_shape=[
            jax.ShapeDtypeStruct((HEADS, T, N, OUT_C), _f32),
            jax.ShapeDtypeStruct((T, N, 32), _f32),
            jax.ShapeDtypeStruct((T, N, 32), _f32),
            jax.ShapeDtypeStruct((HEADS, N, OUT_C), _f32),
        ],
    )(x, W, attA, attB, bias2)


_B_CHUNK = 128
_B_CHUNKS = EP // 32 // _B_CHUNK


def _b_set():
    return [
        pltpu.VMEM((_B_CHUNK,), _i32),
        pltpu.VMEM((_B_CHUNK,), _i32),
        pltpu.VMEM((_B_CHUNK,), _i32),
        pltpu.VMEM((_B_CHUNK,), _i32),
        pltpu.VMEM((_B_CHUNK,), _i32),
        pltpu.VMEM((_B_CHUNK, 32), _f32),
        pltpu.VMEM((_B_CHUNK, 32), _f32),
        pltpu.VMEM((HEADS, _B_CHUNK), _f32),
        pltpu.VMEM((_B_CHUNK, 16), _f32),
        pltpu.SemaphoreType.DMA,
        pltpu.SemaphoreType.DMA,
        pltpu.SemaphoreType.DMA,
    ]


@functools.cache
def _phase_b_built():
    mesh = plsc.VectorSubcoreMesh(core_axis_name="c", subcore_axis_name="s")
    return functools.partial(
        pl.kernel, mesh=mesh,
        compiler_params=pltpu.CompilerParams(
            needs_layout_passes=False, use_tc_tiling_on_sc=False),
        out_type=[
            jax.ShapeDtypeStruct((EP // 128, HEADS, 128), _f32),
            jax.ShapeDtypeStruct((2, DN, 16), _f32),
        ],
        scratch_types=_b_set() + _b_set() + [
            pltpu.VMEM((16, 16), _f32),
            pltpu.VMEM((DN // 16, 16), _f32),
            pltpu.VMEM_SHARED((DN, 16), _f32),
        ],
    )(_phase_b)


def _phase_b(src_hbm, dst_hbm, typ_hbm, sd_hbm, ss_hbm, probs_hbm,
             ex_hbm, denom_hbm, *refs):
    sets = (refs[0:12], refs[12:24])
    probv, zbuf, denom_sh = refs[24], refs[25], refs[26]
    cid = lax.axis_index("c")
    sid = lax.axis_index("s")
    w = cid * 16 + sid
    z16 = jnp.zeros((16,), _f32)
    rows_per_tile = DN // 16
    wbase = w * (_B_CHUNK * _B_CHUNKS)
    iota16 = lax.iota(_i32, 16)

    def zrow(i, _):
        zbuf[i] = z16
        return _
    lax.fori_loop(0, rows_per_tile, zrow, None)
    pltpu.sync_copy(zbuf, denom_sh.at[pl.ds(sid * rows_per_tile, rows_per_tile)])
    for b in range(2):
        scat = sets[b][8]
        for i in range(_B_CHUNK):
            scat[i] = z16
    pltpu.sync_copy(probs_hbm, probv)
    plsc.subcore_barrier()

    def issue_loads(i, b):
        srcv, dstv, typv = sets[b][0:3]
        sem_l = sets[b][9]
        base = wbase + jnp.minimum(i, _B_CHUNKS - 1) * _B_CHUNK
        pltpu.async_copy(src_hbm.at[pl.ds(base, _B_CHUNK)], srcv, sem_l)
        pltpu.async_copy(dst_hbm.at[pl.ds(base, _B_CHUNK)], dstv, sem_l)
        pltpu.async_copy(typ_hbm.at[pl.ds(base, _B_CHUNK)], typv, sem_l)

    def wait_loads(b):
        srcv, dstv, typv = sets[b][0:3]
        sem_l = sets[b][9]
        for buf in (srcv, dstv, typv):
            pltpu.make_async_copy(
                src_hbm.at[pl.ds(0, _B_CHUNK)], buf, sem_l).wait()

    def indices(b):
        srcv, dstv, typv, tsrcv, tdstv = sets[b][0:5]
        for g in range(_B_CHUNK // 16):
            sl = pl.ds(g * 16, 16)
            tt = typv[sl] * N
            tsrcv[sl] = tt + srcv[sl]
            tdstv[sl] = tt + dstv[sl]

    def gather_descs(b):
        tsrcv, tdstv, sdb, ssb = sets[b][3:7]
        sem_g = sets[b][10]
        return [(sd_hbm.at[tdstv], sdb, sem_g),
                (ss_hbm.at[tsrcv], ssb, sem_g)]

    def scatter_desc(b):
        tdstv = sets[b][4]
        scat = sets[b][8]
        sem_w = sets[b][11]
        return scat, denom_sh.at[tdstv], sem_w

    def compute(b):
        sdb, ssb, exb, scat = sets[b][5:9]
        pk = [probv[k] for k in range(KC)]
        for g in range(_B_CHUNK // 16):
            rows = iota16 + g * 16
            for h in range(HEADS):
                acc = jnp.zeros((16,), _f32)
                for k in range(KC):
                    col = jnp.full((16,), h * 8 + k, _i32)
                    v = (plsc.load_gather(sdb, [rows, col])
                         + plsc.load_gather(ssb, [rows, col]))
                    v = jnp.maximum(v, 0.2 * v)
                    acc = acc + pk[k] * v
                ev = jnp.exp(acc)
                exb[h, pl.ds(g * 16, 16)] = ev
                plsc.store_scatter(scat, [rows, jnp.full((16,), h, _i32)], ev)

    def step(i, b):
        wait_loads(1 - b)

        @pl.when(i > 0)
        def _drain():
            sc, dd, sem_w = scatter_desc(1 - b)
            pltpu.make_async_copy(sc, dd, sem_w).wait()

        indices(1 - b)
        for s, d, sem in gather_descs(1 - b):
            pltpu.async_copy(s, d, sem)
        issue_loads(i + 2, b)
        for s, d, sem in gather_descs(b):
            pltpu.make_async_copy(s, d, sem).wait()
        compute(b)
        exb = sets[b][7]
        gc = wbase // 128 + jnp.minimum(i, _B_CHUNKS - 1)
        pltpu.sync_copy(exb, ex_hbm.at[gc])
        sc, dd, sem_w = scatter_desc(b)
        pltpu.async_copy(sc, dd, sem_w, add=True)

    issue_loads(0, 0)
    wait_loads(0)
    indices(0)
    for s, d, sem in gather_descs(0):
        pltpu.async_copy(s, d, sem)
    issue_loads(1, 1)

    def body(j, _):
        step(2 * j, 0)
        step(2 * j + 1, 1)
        return _
    lax.fori_loop(0, _B_CHUNKS // 2, body, None)

    for s, d, sem in gather_descs(0):
        pltpu.make_async_copy(s, d, sem).wait()
    wait_loads(1)
    sc, dd, sem_w = scatter_desc(1)
    pltpu.make_async_copy(sc, dd, sem_w).wait()

    plsc.subcore_barrier()
    pltpu.sync_copy(denom_sh.at[pl.ds(sid * rows_per_tile, rows_per_tile)],
                    denom_hbm.at[cid, pl.ds(sid * rows_per_tile, rows_per_tile)])


def _phase_c_body(d_ref, imp_ref, rec_ref):
    rec_ref[...] = imp_ref[...] / (d_ref[0] + d_ref[1] + 1e-16)


def _phase_c(denom_part, imp_col):
    blk = 2560
    return pl.pallas_call(
        _phase_c_body,
        grid=(DN // blk,),
        in_specs=[
            pl.BlockSpec((2, blk, 16), lambda i: (0, i, 0)),
            pl.BlockSpec((blk, 1), lambda i: (i, 0)),
        ],
        out_specs=pl.BlockSpec((blk, 16), lambda i: (i, 0)),
        out_shape=jax.ShapeDtypeStruct((DN, 16), _f32),
    )(denom_part, imp_col)


def _c2_set():
    return [
        pltpu.VMEM((_B_CHUNK,), _i32),
        pltpu.VMEM((_B_CHUNK,), _i32),
        pltpu.VMEM((_B_CHUNK,), _i32),
        pltpu.VMEM((_B_CHUNK, 16), _f32),
        pltpu.VMEM((HEADS, _B_CHUNK), _f32),
        pltpu.VMEM((HEADS, 8, 16), _f32),
        pltpu.SemaphoreType.DMA,
        pltpu.SemaphoreType.DMA,
    ]


@functools.cache
def _phase_c2_built():
    mesh = plsc.VectorSubcoreMesh(core_axis_name="c", subcore_axis_name="s")
    return functools.partial(
        pl.kernel, mesh=mesh,
        compiler_params=pltpu.CompilerParams(
            needs_layout_passes=False, use_tc_tiling_on_sc=False),
        out_type=jax.ShapeDtypeStruct((EP // 128, HEADS, 8, 16), _f32),
        scratch_types=_c2_set() + _c2_set(),
    )(_phase_c2)


def _phase_c2(dst_hbm, typ_hbm, rec_hbm, ex_hbm, c_hbm, *refs):
    sets = (refs[0:8], refs[8:16])
    cid = lax.axis_index("c")
    sid = lax.axis_index("s")
    w = cid * 16 + sid
    wbase = w * (_B_CHUNK * _B_CHUNKS)
    iota16 = lax.iota(_i32, 16)

    def issue_loads(i, b):
        dstv, typv = sets[b][0:2]
        sem_l = sets[b][6]
        base = wbase + jnp.minimum(i, _B_CHUNKS - 1) * _B_CHUNK
        pltpu.async_copy(dst_hbm.at[pl.ds(base, _B_CHUNK)], dstv, sem_l)
        pltpu.async_copy(typ_hbm.at[pl.ds(base, _B_CHUNK)], typv, sem_l)

    def wait_loads(b):
        dstv, typv = sets[b][0:2]
        sem_l = sets[b][6]
        for buf in (dstv, typv):
            pltpu.make_async_copy(
                dst_hbm.at[pl.ds(0, _B_CHUNK)], buf, sem_l).wait()

    def indices(b):
        dstv, typv, tdstv = sets[b][0:3]
        for g in range(_B_CHUNK // 16):
            sl = pl.ds(g * 16, 16)
            tdstv[sl] = typv[sl] * N + dstv[sl]

    def gather_descs(i, b):
        tdstv, recb, exb = sets[b][2:5]
        sem_g = sets[b][7]
        gc = wbase // 128 + jnp.minimum(i, _B_CHUNKS - 1)
        return [(rec_hbm.at[tdstv], recb, sem_g),
                (ex_hbm.at[gc], exb, sem_g)]

    def compute_write(i, b):
        recb, exb, cb = sets[b][3:6]
        for g in range(_B_CHUNK // 16):
            rows = iota16 + g * 16
            sl = pl.ds(g * 16, 16)
            for h in range(HEADS):
                rc = plsc.load_gather(recb, [rows, jnp.full((16,), h, _i32)])
                cb[h, g] = exb[h, sl] * rc
        gc = wbase // 128 + jnp.minimum(i, _B_CHUNKS - 1)
        pltpu.sync_copy(cb, c_hbm.at[gc])

    def step(i, b):
        wait_loads(1 - b)
        indices(1 - b)
        for sd_, dd_, sem in gather_descs(i + 1, 1 - b):
            pltpu.async_copy(sd_, dd_, sem)
        issue_loads(i + 2, b)
        for sd_, dd_, sem in gather_descs(i, b):
            pltpu.make_async_copy(sd_, dd_, sem).wait()
        compute_write(i, b)

    issue_loads(0, 0)
    wait_loads(0)
    indices(0)
    for sd_, dd_, sem in gather_descs(0, 0):
        pltpu.async_copy(sd_, dd_, sem)
    issue_loads(1, 1)

    def body(j, _):
        step(2 * j, 0)
        step(2 * j + 1, 1)
        return _
    lax.fori_loop(0, _B_CHUNKS // 2, body, None)

    for sd_, dd_, sem in gather_descs(_B_CHUNKS, 0):
        pltpu.make_async_copy(sd_, dd_, sem).wait()
    wait_loads(1)


_DS = 128
_DR = _DS // 128
_D_NSC = EP // 16 // _DS
_ACC_ROWS = N + 16
_NPT = N // 16


def _d_set():
    return [
        pltpu.VMEM((_DS,), _i32),
        pltpu.VMEM((_DS,), _i32),
        pltpu.VMEM((_DS,), _i32),
        pltpu.VMEM((_DR, 128), _i32),
        pltpu.VMEM((_DR, 128), _i32),
        pltpu.VMEM((_DS, OUT_C), _f32),
        pltpu.VMEM((_DS // 16, 16), _f32),
        pltpu.SemaphoreType.DMA,
        pltpu.SemaphoreType.DMA,
        pltpu.SemaphoreType.DMA,
    ]


@functools.cache
def _phase_d_built():
    mesh = plsc.VectorSubcoreMesh(core_axis_name="c", subcore_axis_name="s")
    return functools.partial(
        pl.kernel, mesh=mesh,
        compiler_params=pltpu.CompilerParams(
            needs_layout_passes=False, use_tc_tiling_on_sc=False),
        out_type=jax.ShapeDtypeStruct((HEADS, N, OUT_C), _f32),
        scratch_types=_d_set() + _d_set() + [
            pltpu.VMEM_SHARED((_ACC_ROWS, OUT_C), _f32),
        ],
    )(_phase_d)


def _phase_d(src_hbm, dst_hbm, typ_hbm, xtg_hbm, c_hbm, init_hbm,
             out_hbm, *refs):
    sets = (refs[0:10], refs[10:20])
    acc_sh = refs[20]
    cid = lax.axis_index("c")
    sid = lax.axis_index("s")
    iota16 = lax.iota(_i32, 16)
    hbase = sid * (_DS * _D_NSC)

    for hh in range(2):
        h = cid * 2 + hh
        pltpu.sync_copy(init_hbm.at[h, pl.ds(sid * _NPT, _NPT)],
                        acc_sh.at[pl.ds(sid * _NPT, _NPT)])
        plsc.subcore_barrier()

        def issue_loads(i, b):
            srcv, dstv, typv = sets[b][0:3]
            sem_l = sets[b][7]
            base = hbase + jnp.minimum(i, _D_NSC - 1) * _DS
            pltpu.async_copy(src_hbm.at[pl.ds(base, _DS)], srcv, sem_l)
            pltpu.async_copy(dst_hbm.at[pl.ds(base, _DS)], dstv, sem_l)
            pltpu.async_copy(typ_hbm.at[pl.ds(base, _DS)], typv, sem_l)

        def wait_loads(b):
            srcv, dstv, typv = sets[b][0:3]
            sem_l = sets[b][7]
            for buf in (srcv, dstv, typv):
                pltpu.make_async_copy(
                    src_hbm.at[pl.ds(0, _DS)], buf, sem_l).wait()

        def indices(b):
            srcv, dstv, typv, xsrc2, dst2 = sets[b][0:5]
            for g in range(_DS // 16):
                sl = pl.ds(g * 16, 16)
                r = g // 8
                cs = pl.ds((g % 8) * 16, 16)
                d16 = dstv[sl]
                xsrc2[r, cs] = typv[sl] * N + srcv[sl] + h * (T * N)
                dst2[r, cs] = d16

        def gather_descs(i, b):
            xsrc2 = sets[b][3]
            xtb, cv = sets[b][5:7]
            sem_g = sets[b][8]
            out = []
            for r in range(_DR):
                rs = pl.ds(r * 128, 128)
                out.append((xtg_hbm.at[xsrc2.at[r]], xtb.at[rs], sem_g))
            gc = (hbase + jnp.minimum(i, _D_NSC - 1) * _DS) // 128
            out.append((c_hbm.at[gc, h], cv, sem_g))
            return out

        def scatter_descs(b):
            dst2 = sets[b][4]
            xtb = sets[b][5]
            sem_s = sets[b][9]
            return [(xtb.at[pl.ds(r * 128, 128)], acc_sh.at[dst2.at[r]])
                    for r in range(_DR)], sem_s

        def compute(b):
            xtb, cv = sets[b][5:7]

            def escale(e2, _c):
                for u in range(2):
                    e = e2 * 2 + u
                    ge = jnp.full((16,), e // 16, _i32)
                    le = jnp.full((16,), e % 16, _i32)
                    ce = plsc.load_gather(cv, [ge, le])
                    for q in range(OUT_C // 16):
                        qs = pl.ds(q * 16, 16)
                        xtb[e, qs] = xtb[e, qs] * ce
                return _c
            lax.fori_loop(0, _DS // 2, escale, None)

        def step(i, b):
            wait_loads(1 - b)

            @pl.when(i > 0)
            def _drain():
                descs, sem_s = scatter_descs(1 - b)
                for s, d in descs:
                    pltpu.make_async_copy(s, d, sem_s).wait()

            indices(1 - b)
            for s, d, sem in gather_descs(i + 1, 1 - b):
                pltpu.async_copy(s, d, sem)
            issue_loads(i + 2, b)
            for s, d, sem in gather_descs(i, b):
                pltpu.make_async_copy(s, d, sem).wait()
            compute(b)
            descs, sem_s = scatter_descs(b)
            for s, d in descs:
                pltpu.async_copy(s, d, sem_s, add=True)

        issue_loads(0, 0)
        wait_loads(0)
        indices(0)
        for s, d, sem in gather_descs(0, 0):
            pltpu.async_copy(s, d, sem)
        issue_loads(1, 1)

        def body(j, _):
            step(2 * j, 0)
            step(2 * j + 1, 1)
            return _
        lax.fori_loop(0, _D_NSC // 2, body, None)

        for s, d, sem in gather_descs(_D_NSC, 0):
            pltpu.make_async_copy(s, d, sem).wait()
        wait_loads(1)
        descs, sem_s = scatter_descs(1)
        for s, d in descs:
            pltpu.make_async_copy(s, d, sem_s).wait()

        plsc.subcore_barrier()
        pltpu.sync_copy(acc_sh.at[pl.ds(sid * _NPT, _NPT)],
                        out_hbm.at[h, pl.ds(sid * _NPT, _NPT)])
        plsc.subcore_barrier()


def kernel(x, edge_index, edge_types, W, att_vectors, confounder_probs,
           edge_importance, bias):
    probs = jax.nn.softmax(confounder_probs)
    probs16 = jnp.broadcast_to(
        jnp.zeros((16,), _f32).at[:KC].set(probs)[:, None], (16, 16))

    attA = jnp.zeros((HEADS, OUT_C, 8), _f32).at[:, :, :KC].set(
        jnp.transpose(att_vectors[:, :, :OUT_C], (1, 2, 0)))
    attB = jnp.zeros((HEADS, OUT_C, 8), _f32).at[:, :, :KC].set(
        jnp.transpose(att_vectors[:, :, OUT_C:], (1, 2, 0)))
    bias2 = bias.reshape(1, HOC)

    xtg, sd3, ss3, out_init = _phase_a(x, W, attA, attB, bias2)
    xtg2 = xtg.reshape(HEADS * T * N, OUT_C)
    sd2 = jnp.pad(sd3.reshape(T * N, 32), ((0, DN - T * N), (0, 0)))
    ss2 = jnp.pad(ss3.reshape(T * N, 32), ((0, DN - T * N), (0, 0)))

    srcP = jnp.concatenate([edge_index[0], jnp.zeros((PAD,), _i32)])
    dstP = jnp.concatenate([edge_index[1], jnp.full((PAD,), N, _i32)])
    typP = jnp.concatenate([edge_types, jnp.full((PAD,), T - 1, _i32)])

    ex, denom_part = _phase_b_built()(srcP, dstP, typP, sd2, ss2, probs16)

    imp_col = jnp.concatenate(
        [jnp.repeat(edge_importance, N), jnp.ones((DN - T * N,), _f32)]
    ).reshape(DN, 1)
    rec = _phase_c(denom_part, imp_col)
    cmat = _phase_c2_built()(dstP, typP, rec, ex)

    out_hd = _phase_d_built()(srcP, dstP, typP, xtg2, cmat, out_init)
    return out_hd.transpose(1, 0, 2).reshape(N, HOC)
